# Initial kernel scaffold; baseline (speedup 1.0000x reference)
import functools

import jax
import jax.numpy as jnp
from jax import lax
from jax.experimental import pallas as pl
from jax.experimental.pallas import tpu as pltpu
from jax.experimental.pallas import tpu_sc as plsc

N = 10000
D = 128
OUT = 128
E = 320000

DP = 144
NC = 2
NS = 16
NW = NC * NS
K = 128
ROWS_PER_TILE = 632
N_PAD = ROWS_PER_TILE * NS
CHUNKS = 79
E_PAD = NW * K * CHUNKS
EPS = 1e-5


BN1 = 1000


def _mlp1_body(x_ref, w1a_ref, w1b_ref, out_ref):
    h1 = jnp.maximum(
        jnp.dot(x_ref[...], w1a_ref[...], preferred_element_type=jnp.float32), 0.0)
    h = jnp.dot(h1, w1b_ref[...], preferred_element_type=jnp.float32)
    lane = lax.broadcasted_iota(jnp.int32, (BN1, DP - D), 1)
    aug = jnp.where(lane == 0, 1.0, 0.0).astype(jnp.float32)
    out_ref[...] = jnp.concatenate([h, aug], axis=1)


def _mlp1(x, w1a, w1b):
    return pl.pallas_call(
        _mlp1_body,
        grid=(N // BN1,),
        in_specs=[
            pl.BlockSpec((BN1, D), lambda i: (i, 0)),
            pl.BlockSpec((D, D), lambda i: (0, 0)),
            pl.BlockSpec((D, D), lambda i: (0, 0)),
        ],
        out_specs=pl.BlockSpec((BN1, DP), lambda i: (i, 0)),
        out_shape=jax.ShapeDtypeStruct((N, DP), jnp.float32),
    )(x, w1a, w1b)


def _sc_agg_body(h_hbm, row_hbm, col_hbm, out_hbm, rowi_v, coli_v, rows_v,
                 acc_sh, sem):
    cid = lax.axis_index("c")
    sid = lax.axis_index("s")
    wid = sid * NC + cid

    def zero_body(t, _):
        i = t // (DP // 16)
        j = t % (DP // 16)
        rows_v[i, pl.ds(j * 16, 16)] = jnp.zeros((16,), jnp.float32)
        return 0
    lax.fori_loop(0, K * (DP // 16), zero_body, 0)

    obase = sid * ROWS_PER_TILE
    nfull = ROWS_PER_TILE // K
    rem = ROWS_PER_TILE - nfull * K
    for r in range(nfull):
        pltpu.sync_copy(rows_v, acc_sh.at[pl.ds(obase + r * K, K)])
    pltpu.sync_copy(rows_v.at[pl.ds(0, rem)],
                    acc_sh.at[pl.ds(obase + nfull * K, rem)])

    plsc.subcore_barrier()

    e_per_w = K * CHUNKS

    def chunk_body(c, _):
        base = wid * e_per_w + c * K
        pltpu.sync_copy(row_hbm.at[pl.ds(base, K)], rowi_v)
        pltpu.sync_copy(col_hbm.at[pl.ds(base, K)], coli_v)
        pltpu.async_copy(h_hbm.at[rowi_v], rows_v, sem).wait()
        pltpu.sync_copy(rows_v, acc_sh.at[coli_v], add=True)
        return 0
    lax.fori_loop(0, CHUNKS, chunk_body, 0)

    plsc.subcore_barrier()

    for r in range(nfull):
        pltpu.sync_copy(acc_sh.at[pl.ds(obase + r * K, K)], rows_v)
        pltpu.sync_copy(rows_v, out_hbm.at[cid, pl.ds(obase + r * K, K)])
    pltpu.sync_copy(acc_sh.at[pl.ds(obase + nfull * K, rem)],
                    rows_v.at[pl.ds(0, rem)])
    pltpu.sync_copy(rows_v.at[pl.ds(0, rem)],
                    out_hbm.at[cid, pl.ds(obase + nfull * K, rem)])


def _sc_agg(h_aug, row_p, col_p):
    mesh = plsc.VectorSubcoreMesh(core_axis_name="c", subcore_axis_name="s")
    fn = functools.partial(
        pl.kernel,
        mesh=mesh,
        out_type=jax.ShapeDtypeStruct((NC, N_PAD, DP), jnp.float32),
        scratch_types=[
            pltpu.VMEM((K,), jnp.int32),
            pltpu.VMEM((K,), jnp.int32),
            pltpu.VMEM((K, DP), jnp.float32),
            pltpu.VMEM_SHARED((N_PAD, DP), jnp.float32),
            pltpu.SemaphoreType.DMA,
        ],
    )(_sc_agg_body)
    return fn(h_aug, row_p, col_p)


BN2 = 1000


def _ln(v, g, b):
    mu = jnp.mean(v, axis=-1, keepdims=True)
    var = jnp.mean((v - mu) ** 2, axis=-1, keepdims=True)
    return (v - mu) * lax.rsqrt(var + EPS) * g + b


def _mlp2_body(x_ref, a0_ref, a1_ref, c0_ref, c1_ref, w2f_ref, w2g_ref,
               w2b_ref, g1_ref, b1_ref, g2_ref, b2_ref, w_ref, out_ref):
    agg_sum = a0_ref[...] + a1_ref[...]
    cnt = jnp.maximum(c0_ref[...] + c1_ref[...], 1.0)
    agg = agg_sum / cnt
    ln1 = _ln(agg, g1_ref[...], b1_ref[...])
    fx = x_ref[...] + (x_ref[...] - ln1) * w_ref[...]
    ln2 = _ln(fx, g2_ref[...], b2_ref[...])
    t = jnp.maximum(
        jnp.dot(ln2, w2f_ref[...], preferred_element_type=jnp.float32)
        + jnp.dot(ln1, w2g_ref[...], preferred_element_type=jnp.float32), 0.0)
    out_ref[...] = jnp.dot(t, w2b_ref[...], preferred_element_type=jnp.float32)


def _mlp2(x, a0, a1, c0, c1, w2f, w2g, w2b, g1, b1, g2, b2, w):
    row_spec = pl.BlockSpec((BN2, D), lambda i: (i, 0))
    one_spec = pl.BlockSpec((1, D), lambda i: (0, 0))
    mat_spec = pl.BlockSpec((D, D), lambda i: (0, 0))
    return pl.pallas_call(
        _mlp2_body,
        grid=(N // BN2,),
        in_specs=[
            row_spec, row_spec, row_spec,
            pl.BlockSpec((BN2, 1), lambda i: (i, 0)),
            pl.BlockSpec((BN2, 1), lambda i: (i, 0)),
            mat_spec, mat_spec,
            pl.BlockSpec((D, OUT), lambda i: (0, 0)),
            one_spec, one_spec, one_spec, one_spec, one_spec,
        ],
        out_specs=pl.BlockSpec((BN2, OUT), lambda i: (i, 0)),
        out_shape=jax.ShapeDtypeStruct((N, OUT), jnp.float32),
    )(x, a0, a1, c0, c1, w2f, w2g, w2b, g1, b1, g2, b2, w)


@jax.jit
def kernel(x, edge_index, batch, W1a, W1b, W2a, W2b, w, g1, b1, g2, b2):
    row = edge_index[0]
    col = edge_index[1]
    pad = E_PAD - E
    row_p = jnp.concatenate([row, jnp.zeros((pad,), jnp.int32)])
    col_p = jnp.concatenate([col, jnp.full((pad,), N, jnp.int32)])

    h_aug = _mlp1(x, W1a, W1b)
    parts = _sc_agg(h_aug, row_p, col_p)

    a0 = parts[0, :N, :D]
    a1 = parts[1, :N, :D]
    c0 = parts[0, :N, D:D + 1]
    c1 = parts[1, :N, D:D + 1]

    out = _mlp2(
        x, a0, a1, c0, c1,
        W2a[:D], W2a[D:], W2b,
        g1.reshape(1, D), b1.reshape(1, D),
        g2.reshape(1, D), b2.reshape(1, D), w.reshape(1, D),
    )
    return out


# speedup vs baseline: 4.3038x; 4.3038x over previous
import functools

import jax
import jax.numpy as jnp
from jax import lax
from jax.experimental import pallas as pl
from jax.experimental.pallas import tpu as pltpu
from jax.experimental.pallas import tpu_sc as plsc

N = 10000
D = 128
OUT = 128
E = 320000

DP = 144
NC = 2
NS = 16
NW = NC * NS
K = 128
ROWS_PER_TILE = 632
N_PAD = ROWS_PER_TILE * NS
CHUNKS = 79
E_PAD = NW * K * CHUNKS
EPS = 1e-5


BN1 = 1000


def _mlp1_body(x_ref, w1a_ref, w1b_ref, out_ref):
    h1 = jnp.maximum(
        jnp.dot(x_ref[...], w1a_ref[...], preferred_element_type=jnp.float32), 0.0)
    h = jnp.dot(h1, w1b_ref[...], preferred_element_type=jnp.float32)
    lane = lax.broadcasted_iota(jnp.int32, (BN1, DP - D), 1)
    aug = jnp.where(lane == 0, 1.0, 0.0).astype(jnp.float32)
    out_ref[...] = jnp.concatenate([h, aug], axis=1)


def _mlp1(x, w1a, w1b):
    return pl.pallas_call(
        _mlp1_body,
        grid=(N // BN1,),
        in_specs=[
            pl.BlockSpec((BN1, D), lambda i: (i, 0)),
            pl.BlockSpec((D, D), lambda i: (0, 0)),
            pl.BlockSpec((D, D), lambda i: (0, 0)),
        ],
        out_specs=pl.BlockSpec((BN1, DP), lambda i: (i, 0)),
        out_shape=jax.ShapeDtypeStruct((N, DP), jnp.float32),
    )(x, w1a, w1b)


def _sc_agg_body(h_hbm, row_hbm, col_hbm, out_hbm, rowi_v, coli_v, rows_v,
                 acc_sh, sem):
    cid = lax.axis_index("c")
    sid = lax.axis_index("s")
    wid = sid * NC + cid

    def zero_body(t, _):
        i = t // (DP // 16)
        j = t % (DP // 16)
        rows_v[i, pl.ds(j * 16, 16)] = jnp.zeros((16,), jnp.float32)
        return 0
    lax.fori_loop(0, K * (DP // 16), zero_body, 0)

    obase = sid * ROWS_PER_TILE
    nfull = ROWS_PER_TILE // K
    rem = ROWS_PER_TILE - nfull * K
    for r in range(nfull):
        pltpu.sync_copy(rows_v, acc_sh.at[pl.ds(obase + r * K, K)])
    pltpu.sync_copy(rows_v.at[pl.ds(0, rem)],
                    acc_sh.at[pl.ds(obase + nfull * K, rem)])

    plsc.subcore_barrier()

    e_per_w = K * CHUNKS

    def chunk_body(c, _):
        base = wid * e_per_w + c * K
        pltpu.sync_copy(row_hbm.at[pl.ds(base, K)], rowi_v)
        pltpu.sync_copy(col_hbm.at[pl.ds(base, K)], coli_v)
        pltpu.async_copy(h_hbm.at[rowi_v], rows_v, sem).wait()
        pltpu.sync_copy(rows_v, acc_sh.at[coli_v], add=True)
        return 0
    lax.fori_loop(0, CHUNKS, chunk_body, 0)

    plsc.subcore_barrier()

    for r in range(nfull):
        pltpu.sync_copy(acc_sh.at[pl.ds(obase + r * K, K)], rows_v)
        pltpu.sync_copy(rows_v, out_hbm.at[cid, pl.ds(obase + r * K, K)])
    pltpu.sync_copy(acc_sh.at[pl.ds(obase + nfull * K, rem)],
                    rows_v.at[pl.ds(0, rem)])
    pltpu.sync_copy(rows_v.at[pl.ds(0, rem)],
                    out_hbm.at[cid, pl.ds(obase + nfull * K, rem)])


def _sc_agg(h_aug, row_p, col_p):
    mesh = plsc.VectorSubcoreMesh(core_axis_name="c", subcore_axis_name="s")
    fn = functools.partial(
        pl.kernel,
        mesh=mesh,
        out_type=jax.ShapeDtypeStruct((NC, N_PAD, DP), jnp.float32),
        scratch_types=[
            pltpu.VMEM((K,), jnp.int32),
            pltpu.VMEM((K,), jnp.int32),
            pltpu.VMEM((K, DP), jnp.float32),
            pltpu.VMEM_SHARED((N_PAD, DP), jnp.float32),
            pltpu.SemaphoreType.DMA,
        ],
        compiler_params=pltpu.CompilerParams(use_tc_tiling_on_sc=False),
    )(_sc_agg_body)
    return fn(h_aug, row_p, col_p)


BN2 = 1000


def _ln(v, g, b):
    mu = jnp.mean(v, axis=-1, keepdims=True)
    var = jnp.mean((v - mu) ** 2, axis=-1, keepdims=True)
    return (v - mu) * lax.rsqrt(var + EPS) * g + b


def _mlp2_body(x_ref, a0_ref, a1_ref, c0_ref, c1_ref, w2f_ref, w2g_ref,
               w2b_ref, g1_ref, b1_ref, g2_ref, b2_ref, w_ref, out_ref):
    agg_sum = a0_ref[...] + a1_ref[...]
    cnt = jnp.maximum(c0_ref[...] + c1_ref[...], 1.0)
    agg = agg_sum / cnt
    ln1 = _ln(agg, g1_ref[...], b1_ref[...])
    fx = x_ref[...] + (x_ref[...] - ln1) * w_ref[...]
    ln2 = _ln(fx, g2_ref[...], b2_ref[...])
    t = jnp.maximum(
        jnp.dot(ln2, w2f_ref[...], preferred_element_type=jnp.float32)
        + jnp.dot(ln1, w2g_ref[...], preferred_element_type=jnp.float32), 0.0)
    out_ref[...] = jnp.dot(t, w2b_ref[...], preferred_element_type=jnp.float32)


def _mlp2(x, a0, a1, c0, c1, w2f, w2g, w2b, g1, b1, g2, b2, w):
    row_spec = pl.BlockSpec((BN2, D), lambda i: (i, 0))
    one_spec = pl.BlockSpec((1, D), lambda i: (0, 0))
    mat_spec = pl.BlockSpec((D, D), lambda i: (0, 0))
    return pl.pallas_call(
        _mlp2_body,
        grid=(N // BN2,),
        in_specs=[
            row_spec, row_spec, row_spec,
            pl.BlockSpec((BN2, 1), lambda i: (i, 0)),
            pl.BlockSpec((BN2, 1), lambda i: (i, 0)),
            mat_spec, mat_spec,
            pl.BlockSpec((D, OUT), lambda i: (0, 0)),
            one_spec, one_spec, one_spec, one_spec, one_spec,
        ],
        out_specs=pl.BlockSpec((BN2, OUT), lambda i: (i, 0)),
        out_shape=jax.ShapeDtypeStruct((N, OUT), jnp.float32),
    )(x, a0, a1, c0, c1, w2f, w2g, w2b, g1, b1, g2, b2, w)


@jax.jit
def kernel(x, edge_index, batch, W1a, W1b, W2a, W2b, w, g1, b1, g2, b2):
    row = edge_index[0]
    col = edge_index[1]
    pad = E_PAD - E
    row_p = jnp.concatenate([row, jnp.zeros((pad,), jnp.int32)])
    col_p = jnp.concatenate([col, jnp.full((pad,), N, jnp.int32)])

    h_aug = _mlp1(x, W1a, W1b)
    parts = _sc_agg(h_aug, row_p, col_p)

    a0 = parts[0, :N, :D]
    a1 = parts[1, :N, :D]
    c0 = parts[0, :N, D:D + 1]
    c1 = parts[1, :N, D:D + 1]

    out = _mlp2(
        x, a0, a1, c0, c1,
        W2a[:D], W2a[D:], W2b,
        g1.reshape(1, D), b1.reshape(1, D),
        g2.reshape(1, D), b2.reshape(1, D), w.reshape(1, D),
    )
    return out
